# baseline (device time: 109228 ns/iter reference)
import jax
import jax.numpy as jnp
from jax import lax
from jax.experimental import pallas as pl
from jax.experimental.pallas import tpu as pltpu

N_DEV = 4


def kernel(t, W):
    m_per, k = t.shape
    _, n = W.shape
    m_chunk = m_per // N_DEV
    kh = k // 2
    nh = n // 2

    def body(t_ref, w_ref, out_ref,
             wb_ref,
             cw_ref, ccw_ref, agcw_ref, agccw_ref,
             cw_ssem, cw_rsem, ccw_ssem, ccw_rsem,
             agcw_ssem, agcw_rsem, agccw_ssem, agccw_rsem):
        my = lax.axis_index("i")
        left = (my - 1) % N_DEV
        right = (my + 1) % N_DEV

        barrier_sem = pltpu.get_barrier_semaphore()
        for nbr in (left, right):
            pl.semaphore_signal(
                barrier_sem, inc=1,
                device_id=(nbr,), device_id_type=pl.DeviceIdType.MESH,
            )
        pl.semaphore_wait(barrier_sem, 2)

        def hop(buf, ssem, rsem, h, target):
            return pltpu.make_async_remote_copy(
                src_ref=buf.at[h],
                dst_ref=buf.at[h + 1],
                send_sem=ssem.at[h],
                recv_sem=rsem.at[h + 1],
                device_id=(target,),
                device_id_type=pl.DeviceIdType.MESH,
            )

        pending = []

        cw_ref[0, :, :] = t_ref[
            pl.ds(((my - 1) % N_DEV) * m_chunk, m_chunk), pl.ds(0, kh)
        ].astype(jnp.bfloat16)
        ccw_ref[0, :, :] = t_ref[
            pl.ds(((my + 1) % N_DEV) * m_chunk, m_chunk), pl.ds(kh, kh)
        ].astype(jnp.bfloat16)
        rd_cw = hop(cw_ref, cw_ssem, cw_rsem, 0, right)
        rd_ccw = hop(ccw_ref, ccw_ssem, ccw_rsem, 0, left)
        rd_cw.start()
        rd_ccw.start()
        pending += [rd_cw, rd_ccw]

        wb_ref[:, :] = w_ref[:, :].astype(jnp.bfloat16)

        def tb_sub(c, col0):
            return t_ref[
                pl.ds(c * m_chunk, m_chunk), pl.ds(col0, kh)
            ].astype(jnp.bfloat16)

        for h in range(N_DEV - 1):
            rd_cw.wait_recv()
            if h < N_DEV - 2:
                cw_ref[h + 1, :, :] = (
                    cw_ref[h + 1, :, :] + tb_sub((my - 2 - h) % N_DEV, 0)
                )
                rd_cw = hop(cw_ref, cw_ssem, cw_rsem, h + 1, right)
                rd_cw.start()
                pending.append(rd_cw)
            rd_ccw.wait_recv()
            if h < N_DEV - 2:
                ccw_ref[h + 1, :, :] = (
                    ccw_ref[h + 1, :, :] + tb_sub((my + 2 + h) % N_DEV, kh)
                )
                rd_ccw = hop(ccw_ref, ccw_ssem, ccw_rsem, h + 1, left)
                rd_ccw.start()
                pending.append(rd_ccw)

        red = jnp.concatenate(
            [cw_ref[N_DEV - 1, :, :] + tb_sub(my, 0),
             ccw_ref[N_DEV - 1, :, :] + tb_sub(my, kh)],
            axis=1,
        )
        res_cw = jnp.dot(
            red, wb_ref[:, pl.ds(0, nh)], preferred_element_type=jnp.float32
        )
        agcw_ref[0, :, :] = res_cw.astype(jnp.bfloat16)
        ag_cw = hop(agcw_ref, agcw_ssem, agcw_rsem, 0, right)
        ag_cw.start()
        pending.append(ag_cw)

        res_ccw = jnp.dot(
            red, wb_ref[:, pl.ds(nh, nh)], preferred_element_type=jnp.float32
        )
        agccw_ref[0, :, :] = res_ccw.astype(jnp.bfloat16)
        ag_ccw = hop(agccw_ref, agccw_ssem, agccw_rsem, 0, left)
        ag_ccw.start()
        pending.append(ag_ccw)

        out_ref[pl.ds(my * m_chunk, m_chunk), pl.ds(0, nh)] = res_cw
        out_ref[pl.ds(my * m_chunk, m_chunk), pl.ds(nh, nh)] = res_ccw

        for h in range(N_DEV - 1):
            ag_cw.wait_recv()
            if h < N_DEV - 2:
                ag_cw = hop(agcw_ref, agcw_ssem, agcw_rsem, h + 1, right)
                ag_cw.start()
                pending.append(ag_cw)
            out_ref[
                pl.ds(((my - 1 - h) % N_DEV) * m_chunk, m_chunk), pl.ds(0, nh)
            ] = agcw_ref[h + 1, :, :].astype(jnp.float32)

            ag_ccw.wait_recv()
            if h < N_DEV - 2:
                ag_ccw = hop(agccw_ref, agccw_ssem, agccw_rsem, h + 1, left)
                ag_ccw.start()
                pending.append(ag_ccw)
            out_ref[
                pl.ds(((my + 1 + h) % N_DEV) * m_chunk, m_chunk), pl.ds(nh, nh)
            ] = agccw_ref[h + 1, :, :].astype(jnp.float32)

        for rd in pending:
            rd.wait_send()

    return pl.pallas_call(
        body,
        out_shape=jax.ShapeDtypeStruct((m_per, n), jnp.float32),
        in_specs=[
            pl.BlockSpec(memory_space=pltpu.VMEM),
            pl.BlockSpec(memory_space=pltpu.VMEM),
        ],
        out_specs=pl.BlockSpec(memory_space=pltpu.VMEM),
        scratch_shapes=[
            pltpu.VMEM((k, n), jnp.bfloat16),
            pltpu.VMEM((N_DEV, m_chunk, kh), jnp.bfloat16),
            pltpu.VMEM((N_DEV, m_chunk, kh), jnp.bfloat16),
            pltpu.VMEM((N_DEV, m_chunk, nh), jnp.bfloat16),
            pltpu.VMEM((N_DEV, m_chunk, nh), jnp.bfloat16),
            pltpu.SemaphoreType.DMA((N_DEV,)),
            pltpu.SemaphoreType.DMA((N_DEV,)),
            pltpu.SemaphoreType.DMA((N_DEV,)),
            pltpu.SemaphoreType.DMA((N_DEV,)),
            pltpu.SemaphoreType.DMA((N_DEV,)),
            pltpu.SemaphoreType.DMA((N_DEV,)),
            pltpu.SemaphoreType.DMA((N_DEV,)),
            pltpu.SemaphoreType.DMA((N_DEV,)),
        ],
        compiler_params=pltpu.CompilerParams(
            collective_id=0,
            vmem_limit_bytes=100 * 1024 * 1024,
        ),
    )(t, W)


# device time: 97958 ns/iter; 1.1150x vs baseline; 1.1150x over previous
import jax
import jax.numpy as jnp
from jax import lax
from jax.experimental import pallas as pl
from jax.experimental.pallas import tpu as pltpu

N_DEV = 4


def kernel(t, W):
    m_per, k = t.shape
    _, n = W.shape
    m_chunk = m_per // N_DEV
    kh = k // 2
    nh = n // 2

    def body(t_ref, w_ref, out_ref,
             cw_ref, ccw_ref, agcw_ref, agccw_ref,
             cw_ssem, cw_rsem, ccw_ssem, ccw_rsem,
             agcw_ssem, agcw_rsem, agccw_ssem, agccw_rsem):
        my = lax.axis_index("i")
        left = (my - 1) % N_DEV
        right = (my + 1) % N_DEV

        barrier_sem = pltpu.get_barrier_semaphore()
        for nbr in (left, right):
            pl.semaphore_signal(
                barrier_sem, inc=1,
                device_id=(nbr,), device_id_type=pl.DeviceIdType.MESH,
            )
        pl.semaphore_wait(barrier_sem, 2)

        def t_sub(c, col0):
            return t_ref[
                pl.ds(c * m_chunk, m_chunk), pl.ds(col0, kh)
            ].astype(jnp.bfloat16)

        def hop(src, dst, ssem, rsem, slot_s, slot_r, target):
            return pltpu.make_async_remote_copy(
                src_ref=src.at[slot_s],
                dst_ref=dst.at[slot_r],
                send_sem=ssem.at[slot_s],
                recv_sem=rsem.at[slot_r],
                device_id=(target,),
                device_id_type=pl.DeviceIdType.MESH,
            )

        cw_ref[0, :, :] = t_sub((my - 1) % N_DEV, 0)
        ccw_ref[0, :, :] = t_sub((my + 1) % N_DEV, kh)
        for h in range(N_DEV - 1):
            s, r = h % 2, (h + 1) % 2
            rd_cw = hop(cw_ref, cw_ref, cw_ssem, cw_rsem, s, r, right)
            rd_ccw = hop(ccw_ref, ccw_ref, ccw_ssem, ccw_rsem, s, r, left)
            rd_cw.start()
            rd_ccw.start()
            rd_cw.wait()
            rd_ccw.wait()
            if h < N_DEV - 2:
                cw_ref[r, :, :] = cw_ref[r, :, :] + t_sub((my - 2 - h) % N_DEV, 0)
                ccw_ref[r, :, :] = ccw_ref[r, :, :] + t_sub((my + 2 + h) % N_DEV, kh)

        last = (N_DEV - 1) % 2
        red = jnp.concatenate(
            [cw_ref[last, :, :] + t_sub(my, 0),
             ccw_ref[last, :, :] + t_sub(my, kh)],
            axis=1,
        )
        res_cw = jnp.dot(
            red, w_ref[:, pl.ds(0, nh)].astype(jnp.bfloat16),
            preferred_element_type=jnp.float32,
        )
        agcw_ref[0, :, :] = res_cw.astype(jnp.bfloat16)
        rd_cw = hop(agcw_ref, agcw_ref, agcw_ssem, agcw_rsem, 0, 1, right)
        rd_cw.start()
        res_ccw = jnp.dot(
            red, w_ref[:, pl.ds(nh, nh)].astype(jnp.bfloat16),
            preferred_element_type=jnp.float32,
        )
        agccw_ref[0, :, :] = res_ccw.astype(jnp.bfloat16)
        rd_ccw = hop(agccw_ref, agccw_ref, agccw_ssem, agccw_rsem, 0, 1, left)
        rd_ccw.start()
        out_ref[pl.ds(my * m_chunk, m_chunk), pl.ds(0, nh)] = res_cw
        out_ref[pl.ds(my * m_chunk, m_chunk), pl.ds(nh, nh)] = res_ccw

        for h in range(N_DEV - 1):
            s, r = h % 2, (h + 1) % 2
            rd_cw.wait()
            rd_ccw.wait()
            if h < N_DEV - 2:
                rd_cw = hop(agcw_ref, agcw_ref, agcw_ssem, agcw_rsem, r, s, right)
                rd_cw.start()
                rd_ccw = hop(agccw_ref, agccw_ref, agccw_ssem, agccw_rsem, r, s, left)
                rd_ccw.start()
            oc_cw = (my - 1 - h) % N_DEV
            oc_ccw = (my + 1 + h) % N_DEV
            out_ref[pl.ds(oc_cw * m_chunk, m_chunk), pl.ds(0, nh)] = (
                agcw_ref[r, :, :].astype(jnp.float32)
            )
            out_ref[pl.ds(oc_ccw * m_chunk, m_chunk), pl.ds(nh, nh)] = (
                agccw_ref[r, :, :].astype(jnp.float32)
            )

    return pl.pallas_call(
        body,
        out_shape=jax.ShapeDtypeStruct((m_per, n), jnp.float32),
        in_specs=[
            pl.BlockSpec(memory_space=pltpu.VMEM),
            pl.BlockSpec(memory_space=pltpu.VMEM),
        ],
        out_specs=pl.BlockSpec(memory_space=pltpu.VMEM),
        scratch_shapes=[
            pltpu.VMEM((2, m_chunk, kh), jnp.bfloat16),
            pltpu.VMEM((2, m_chunk, kh), jnp.bfloat16),
            pltpu.VMEM((2, m_chunk, nh), jnp.bfloat16),
            pltpu.VMEM((2, m_chunk, nh), jnp.bfloat16),
            pltpu.SemaphoreType.DMA((2,)),
            pltpu.SemaphoreType.DMA((2,)),
            pltpu.SemaphoreType.DMA((2,)),
            pltpu.SemaphoreType.DMA((2,)),
            pltpu.SemaphoreType.DMA((2,)),
            pltpu.SemaphoreType.DMA((2,)),
            pltpu.SemaphoreType.DMA((2,)),
            pltpu.SemaphoreType.DMA((2,)),
        ],
        compiler_params=pltpu.CompilerParams(collective_id=0),
    )(t, W)
